# Initial kernel scaffold; baseline (speedup 1.0000x reference)
import jax
import jax.numpy as jnp
from jax.experimental import pallas as pl


def kernel(z, P):
    raise NotImplementedError("write your pallas kernel here")


# speedup vs baseline: 1.6240x; 1.6240x over previous
import jax
import jax.numpy as jnp
from jax.experimental import pallas as pl

_N = 65536
_D = 512
_K = 512
_R = 1024


def _vq_body(z_ref, p_ref, sp_ref, zhat_ref, zt_ref, idx_ref):
    z = z_ref[...]
    p = p_ref[...]
    sp = sp_ref[...]
    sz = jnp.sum(z * z, axis=1, keepdims=True)
    zp = jax.lax.dot_general(
        z, p, (((1,), (1,)), ((), ())),
        preferred_element_type=jnp.float32)
    d = (sz + sp) - 2.0 * zp
    m = jnp.min(d, axis=1, keepdims=True)
    iota = jax.lax.broadcasted_iota(jnp.int32, d.shape, 1)
    idx = jnp.min(jnp.where(d == m, iota, jnp.int32(_K)),
                  axis=1, keepdims=True)
    onehot = (iota == idx).astype(jnp.float32)
    zt = jax.lax.dot_general(
        onehot, p, (((1,), (0,)), ((), ())),
        preferred_element_type=jnp.float32,
        precision=jax.lax.Precision.HIGHEST)
    zhat_ref[...] = 0.7 * z + 0.3 * zt
    zt_ref[...] = zt
    idx_ref[...] = idx


def kernel(z, P):
    sp = jnp.sum(P * P, axis=1)[None, :]
    nb = _N // _R
    zhat, zt, idx2 = pl.pallas_call(
        _vq_body,
        grid=(nb,),
        in_specs=[
            pl.BlockSpec((_R, _D), lambda b: (b, 0)),
            pl.BlockSpec((_K, _D), lambda b: (0, 0)),
            pl.BlockSpec((1, _K), lambda b: (0, 0)),
        ],
        out_specs=[
            pl.BlockSpec((_R, _D), lambda b: (b, 0)),
            pl.BlockSpec((_R, _D), lambda b: (b, 0)),
            pl.BlockSpec((_R, 1), lambda b: (b, 0)),
        ],
        out_shape=[
            jax.ShapeDtypeStruct((_N, _D), jnp.float32),
            jax.ShapeDtypeStruct((_N, _D), jnp.float32),
            jax.ShapeDtypeStruct((_N, 1), jnp.int32),
        ],
    )(z, P, sp)
    return (zhat, zt, idx2[:, 0])


# speedup vs baseline: 3.0076x; 1.8519x over previous
import jax
import jax.numpy as jnp
from jax.experimental import pallas as pl

_N = 65536
_D = 512
_K = 512
_R = 1024


def _vq_body(z_ref, p_ref, sp_ref, zhat_ref, zt_ref, idx_ref):
    z = z_ref[...]
    p = p_ref[...]
    sp = sp_ref[...]
    sz = jnp.sum(z * z, axis=1, keepdims=True)
    zp = jax.lax.dot_general(
        z, p, (((1,), (1,)), ((), ())),
        preferred_element_type=jnp.float32)
    d = (sz + sp) - 2.0 * zp
    m = jnp.min(d, axis=1, keepdims=True)
    iota = jax.lax.broadcasted_iota(jnp.int32, d.shape, 1)
    idx = jnp.min(jnp.where(d == m, iota, jnp.int32(_K)),
                  axis=1, keepdims=True)
    onehot = (iota == idx).astype(jnp.float32)
    zt = jax.lax.dot_general(
        onehot, p, (((1,), (0,)), ((), ())),
        preferred_element_type=jnp.float32)
    zhat_ref[...] = 0.7 * z + 0.3 * zt
    zt_ref[...] = zt
    idx_ref[...] = idx


def kernel(z, P):
    sp = jnp.sum(P * P, axis=1)[None, :]
    nb = _N // _R
    zhat, zt, idx2 = pl.pallas_call(
        _vq_body,
        grid=(nb,),
        in_specs=[
            pl.BlockSpec((_R, _D), lambda b: (b, 0)),
            pl.BlockSpec((_K, _D), lambda b: (0, 0)),
            pl.BlockSpec((1, _K), lambda b: (0, 0)),
        ],
        out_specs=[
            pl.BlockSpec((_R, _D), lambda b: (b, 0)),
            pl.BlockSpec((_R, _D), lambda b: (b, 0)),
            pl.BlockSpec((_R, 1), lambda b: (b, 0)),
        ],
        out_shape=[
            jax.ShapeDtypeStruct((_N, _D), jnp.float32),
            jax.ShapeDtypeStruct((_N, _D), jnp.float32),
            jax.ShapeDtypeStruct((_N, 1), jnp.int32),
        ],
    )(z, P, sp)
    return (zhat, zt, idx2[:, 0])


# speedup vs baseline: 3.3911x; 1.1275x over previous
import jax
import jax.numpy as jnp
from jax.experimental import pallas as pl

_N = 65536
_D = 512
_K = 512
_R = 2048


def _vq_body(z_ref, p_ref, sp_ref, zhat_ref, zt_ref, idx_ref):
    z = z_ref[...]
    p = p_ref[...]
    sp = sp_ref[...]
    sz = jnp.sum(z * z, axis=1, keepdims=True)
    zp = jax.lax.dot_general(
        z, p, (((1,), (1,)), ((), ())),
        preferred_element_type=jnp.float32)
    d = (sz + sp) - 2.0 * zp
    m = jnp.min(d, axis=1, keepdims=True)
    iota = jax.lax.broadcasted_iota(jnp.int32, d.shape, 1)
    idx = jnp.min(jnp.where(d == m, iota, jnp.int32(_K)),
                  axis=1, keepdims=True)
    onehot = (iota == idx).astype(jnp.float32)
    zt = jax.lax.dot_general(
        onehot, p, (((1,), (0,)), ((), ())),
        preferred_element_type=jnp.float32)
    zhat_ref[...] = 0.7 * z + 0.3 * zt
    zt_ref[...] = zt
    idx_ref[...] = idx


def kernel(z, P):
    sp = jnp.sum(P * P, axis=1)[None, :]
    nb = _N // _R
    zhat, zt, idx2 = pl.pallas_call(
        _vq_body,
        grid=(nb,),
        in_specs=[
            pl.BlockSpec((_R, _D), lambda b: (b, 0)),
            pl.BlockSpec((_K, _D), lambda b: (0, 0)),
            pl.BlockSpec((1, _K), lambda b: (0, 0)),
        ],
        out_specs=[
            pl.BlockSpec((_R, _D), lambda b: (b, 0)),
            pl.BlockSpec((_R, _D), lambda b: (b, 0)),
            pl.BlockSpec((_R, 1), lambda b: (b, 0)),
        ],
        out_shape=[
            jax.ShapeDtypeStruct((_N, _D), jnp.float32),
            jax.ShapeDtypeStruct((_N, _D), jnp.float32),
            jax.ShapeDtypeStruct((_N, 1), jnp.int32),
        ],
    )(z, P, sp)
    return (zhat, zt, idx2[:, 0])
